# Initial kernel scaffold; baseline (speedup 1.0000x reference)
import functools

import jax
import jax.numpy as jnp
from jax import lax
from jax.experimental import pallas as pl
from jax.experimental.pallas import tpu as pltpu
from jax.experimental.pallas import tpu_sc as plsc

B = 16384
L = 50
EMB = 32

NC = 2
NS = 16
NW = NC * NS
ROWS_PER_W = B // NW
CHUNK = 16
NCHUNK = ROWS_PER_W // CHUNK
IDX_W = 100
NGATHER = CHUNK * L // IDX_W

_sc_mesh = plsc.VectorSubcoreMesh(core_axis_name="c", subcore_axis_name="s")


@functools.partial(
    pl.kernel,
    mesh=_sc_mesh,
    out_type=[
        jax.ShapeDtypeStruct((B, EMB), jnp.float32),
        jax.ShapeDtypeStruct((B, EMB), jnp.float32),
    ],
    scratch_types=[
        pltpu.VMEM((NGATHER, IDX_W), jnp.int32),
        pltpu.VMEM((CHUNK * L, EMB), jnp.float32),
        pltpu.VMEM((CHUNK,), jnp.int32),
        pltpu.VMEM((CHUNK, EMB), jnp.float32),
        pltpu.VMEM((CHUNK, EMB), jnp.float32),
        pltpu.SemaphoreType.DMA,
        pltpu.SemaphoreType.DMA,
    ],
)
def _pool_sc(nids2d, click, emb, nid_emb, pooled_hbm, clicked_hbm,
             idx2, rows_v, click_v, yrows_v, pooled_v, gsem, csem):
    wid = lax.axis_index("s") * NC + lax.axis_index("c")
    base = wid * ROWS_PER_W

    def chunk_body(c, carry):
        row0 = base + c * CHUNK
        pltpu.sync_copy(nids2d.at[pl.ds(row0 * L // IDX_W, NGATHER)], idx2)
        pltpu.sync_copy(click.at[pl.ds(row0, CHUNK)], click_v)
        cp_click = pltpu.async_copy(nid_emb.at[click_v], yrows_v, csem)
        cps = [
            pltpu.async_copy(
                emb.at[idx2.at[g]],
                rows_v.at[pl.ds(g * IDX_W, IDX_W)],
                gsem,
            )
            for g in range(NGATHER)
        ]
        for cp in cps:
            cp.wait()

        def row_body(i, carry2):
            k0 = i * L
            acc0 = rows_v[k0, pl.ds(0, 16)]
            acc1 = rows_v[k0, pl.ds(16, 16)]
            for j in range(1, L):
                acc0 = acc0 + rows_v[k0 + j, pl.ds(0, 16)]
                acc1 = acc1 + rows_v[k0 + j, pl.ds(16, 16)]
            pooled_v[i, pl.ds(0, 16)] = acc0
            pooled_v[i, pl.ds(16, 16)] = acc1
            return carry2

        lax.fori_loop(0, CHUNK, row_body, 0)
        cp_click.wait()
        pltpu.sync_copy(pooled_v, pooled_hbm.at[pl.ds(row0, CHUNK)])
        pltpu.sync_copy(yrows_v, clicked_hbm.at[pl.ds(row0, CHUNK)])
        return carry

    lax.fori_loop(0, NCHUNK, chunk_body, 0)


BM = 2048


def _mlp_body(x1_ref, x2_ref, w1_ref, b1_ref, w2_ref, b2_ref,
              w3_ref, b3_ref, w4_ref, b4_ref, y_ref):
    x = jnp.concatenate([x1_ref[...], x2_ref[...]], axis=1)
    dn = (((1,), (1,)), ((), ()))
    h = lax.dot_general(x, w1_ref[...], dn, preferred_element_type=jnp.float32)
    h = jnp.maximum(h + b1_ref[...], 0.0)
    h = lax.dot_general(h, w2_ref[...], dn, preferred_element_type=jnp.float32)
    h = jnp.maximum(h + b2_ref[...], 0.0)
    h = lax.dot_general(h, w3_ref[...], dn, preferred_element_type=jnp.float32)
    h = jnp.maximum(h + b3_ref[...], 0.0)
    h = lax.dot_general(h, w4_ref[...], dn, preferred_element_type=jnp.float32)
    y_ref[...] = jnp.maximum(h + b4_ref[...], 0.0)


def _mlp(x1, x2, W1, b1, W2, b2, W3, b3, W4, b4):
    full = lambda s: pl.BlockSpec(s, lambda i: (0, 0))
    return pl.pallas_call(
        _mlp_body,
        grid=(B // BM,),
        in_specs=[
            pl.BlockSpec((BM, EMB), lambda i: (i, 0)),
            pl.BlockSpec((BM, EMB), lambda i: (i, 0)),
            full(W1.shape), full((1, 256)),
            full(W2.shape), full((1, 256)),
            full(W3.shape), full((1, 128)),
            full(W4.shape), full((1, 2)),
        ],
        out_specs=pl.BlockSpec((BM, 2), lambda i: (i, 0)),
        out_shape=jax.ShapeDtypeStruct((B, 2), jnp.float32),
    )(x1, x2, W1, b1.reshape(1, -1), W2, b2.reshape(1, -1),
      W3, b3.reshape(1, -1), W4, b4.reshape(1, -1))


def kernel(input_nids, click_items, input_emb, nid_emb,
           W1, b1, W2, b2, W3, b3, W4, b4):
    nids2d = input_nids.reshape(B * L // IDX_W, IDX_W)
    pooled, clicked = _pool_sc(nids2d, click_items, input_emb, nid_emb)
    return _mlp(pooled, clicked, W1, b1, W2, b2, W3, b3, W4, b4)


# speedup vs baseline: 1.6539x; 1.6539x over previous
import functools

import jax
import jax.numpy as jnp
from jax import lax
from jax.experimental import pallas as pl
from jax.experimental.pallas import tpu as pltpu
from jax.experimental.pallas import tpu_sc as plsc

B = 16384
L = 50
EMB = 32

NC = 2
NS = 16
NW = NC * NS
ROWS_PER_W = B // NW
CHUNK = 16
NCHUNK = ROWS_PER_W // CHUNK
IDX_W = 100
NGATHER = CHUNK * L // IDX_W

_sc_mesh = plsc.VectorSubcoreMesh(core_axis_name="c", subcore_axis_name="s")


@functools.partial(
    pl.kernel,
    mesh=_sc_mesh,
    out_type=[
        jax.ShapeDtypeStruct((B, EMB), jnp.float32),
        jax.ShapeDtypeStruct((B, EMB), jnp.float32),
    ],
    scratch_types=[
        pltpu.VMEM((NGATHER, IDX_W), jnp.int32),
        pltpu.VMEM((CHUNK * L, EMB), jnp.float32),
        pltpu.VMEM((CHUNK,), jnp.int32),
        pltpu.VMEM((CHUNK, EMB), jnp.float32),
        pltpu.VMEM((CHUNK, EMB), jnp.float32),
        pltpu.SemaphoreType.DMA,
        pltpu.SemaphoreType.DMA,
    ],
    compiler_params=pltpu.CompilerParams(use_tc_tiling_on_sc=False),
)
def _pool_sc(nids2d, click, emb, nid_emb, pooled_hbm, clicked_hbm,
             idx2, rows_v, click_v, yrows_v, pooled_v, gsem, csem):
    wid = lax.axis_index("s") * NC + lax.axis_index("c")
    base = wid * ROWS_PER_W

    def chunk_body(c, carry):
        row0 = pl.multiple_of(base + c * CHUNK, CHUNK)
        pltpu.sync_copy(
            nids2d.at[pl.ds(pl.multiple_of(row0 * L // IDX_W, 8), NGATHER)],
            idx2)
        pltpu.sync_copy(click.at[pl.ds(row0, CHUNK)], click_v)
        cp_click = pltpu.async_copy(nid_emb.at[click_v], yrows_v, csem)
        cps = [
            pltpu.async_copy(
                emb.at[idx2.at[g]],
                rows_v.at[pl.ds(g * IDX_W, IDX_W)],
                gsem,
            )
            for g in range(NGATHER)
        ]
        for cp in cps:
            cp.wait()

        def row_body(i, carry2):
            k0 = i * L
            acc0 = rows_v[k0, pl.ds(0, 16)]
            acc1 = rows_v[k0, pl.ds(16, 16)]
            for j in range(1, L):
                acc0 = acc0 + rows_v[k0 + j, pl.ds(0, 16)]
                acc1 = acc1 + rows_v[k0 + j, pl.ds(16, 16)]
            pooled_v[i, pl.ds(0, 16)] = acc0
            pooled_v[i, pl.ds(16, 16)] = acc1
            return carry2

        lax.fori_loop(0, CHUNK, row_body, 0)
        cp_click.wait()
        pltpu.sync_copy(pooled_v, pooled_hbm.at[pl.ds(row0, CHUNK)])
        pltpu.sync_copy(yrows_v, clicked_hbm.at[pl.ds(row0, CHUNK)])
        return carry

    lax.fori_loop(0, NCHUNK, chunk_body, 0)


BM = 2048


def _mlp_body(x1_ref, x2_ref, w1_ref, b1_ref, w2_ref, b2_ref,
              w3_ref, b3_ref, w4_ref, b4_ref, y_ref):
    x = jnp.concatenate([x1_ref[...], x2_ref[...]], axis=1)
    dn = (((1,), (1,)), ((), ()))
    h = lax.dot_general(x, w1_ref[...], dn, preferred_element_type=jnp.float32)
    h = jnp.maximum(h + b1_ref[...], 0.0)
    h = lax.dot_general(h, w2_ref[...], dn, preferred_element_type=jnp.float32)
    h = jnp.maximum(h + b2_ref[...], 0.0)
    h = lax.dot_general(h, w3_ref[...], dn, preferred_element_type=jnp.float32)
    h = jnp.maximum(h + b3_ref[...], 0.0)
    h = lax.dot_general(h, w4_ref[...], dn, preferred_element_type=jnp.float32)
    y_ref[...] = jnp.maximum(h + b4_ref[...], 0.0)


def _mlp(x1, x2, W1, b1, W2, b2, W3, b3, W4, b4):
    full = lambda s: pl.BlockSpec(s, lambda i: (0, 0))
    return pl.pallas_call(
        _mlp_body,
        grid=(B // BM,),
        in_specs=[
            pl.BlockSpec((BM, EMB), lambda i: (i, 0)),
            pl.BlockSpec((BM, EMB), lambda i: (i, 0)),
            full(W1.shape), full((1, 256)),
            full(W2.shape), full((1, 256)),
            full(W3.shape), full((1, 128)),
            full(W4.shape), full((1, 2)),
        ],
        out_specs=pl.BlockSpec((BM, 2), lambda i: (i, 0)),
        out_shape=jax.ShapeDtypeStruct((B, 2), jnp.float32),
    )(x1, x2, W1, b1.reshape(1, -1), W2, b2.reshape(1, -1),
      W3, b3.reshape(1, -1), W4, b4.reshape(1, -1))


def kernel(input_nids, click_items, input_emb, nid_emb,
           W1, b1, W2, b2, W3, b3, W4, b4):
    nids2d = input_nids.reshape(B * L // IDX_W, IDX_W)
    pooled, clicked = _pool_sc(nids2d, click_items, input_emb, nid_emb)
    return _mlp(pooled, clicked, W1, b1, W2, b2, W3, b3, W4, b4)


# speedup vs baseline: 1.7077x; 1.0325x over previous
import functools

import jax
import jax.numpy as jnp
from jax import lax
from jax.experimental import pallas as pl
from jax.experimental.pallas import tpu as pltpu
from jax.experimental.pallas import tpu_sc as plsc

B = 16384
L = 50
EMB = 32
VOCAB = 1000000

NC = 2
NS = 16
NW = NC * NS
ROWS_PER_W = B // NW
CHUNK = 16
NCHUNK = ROWS_PER_W // CHUNK
IDX_W = 100
NGATHER = CHUNK * L // IDX_W

_sc_mesh = plsc.VectorSubcoreMesh(core_axis_name="c", subcore_axis_name="s")


TCOL = 2048
NBLK = (VOCAB + TCOL - 1) // TCOL
VROWS = NBLK * TCOL


def _transpose_body(x0, x1, x2, x3, o_ref):
    bands = [x.T for x in (x0[...], x1[...], x2[...], x3[...])]
    o_ref[...] = jnp.concatenate(bands, axis=1)


def _to_row_major(table_t):
    last = (VOCAB + TCOL // 4 - 1) // (TCOL // 4) - 1
    spec = lambda q: pl.BlockSpec(
        (EMB, TCOL // 4),
        lambda i, q=q: (0, jnp.minimum(4 * i + q, last)))
    out = pl.pallas_call(
        _transpose_body,
        grid=(NBLK,),
        in_specs=[spec(0), spec(1), spec(2), spec(3)],
        out_specs=pl.BlockSpec((TCOL // 4, 4 * EMB), lambda i: (i, 0)),
        out_shape=jax.ShapeDtypeStruct((VROWS // 4, 4 * EMB), jnp.float32),
    )(table_t, table_t, table_t, table_t)
    return out.reshape(VROWS, EMB)


def _scramble(idx):
    return ((idx >> 11) << 11) + ((idx & 511) << 2) + ((idx & 2047) >> 9)


@functools.partial(
    pl.kernel,
    mesh=_sc_mesh,
    out_type=[
        jax.ShapeDtypeStruct((B, EMB), jnp.float32),
        jax.ShapeDtypeStruct((B, EMB), jnp.float32),
    ],
    scratch_types=[
        pltpu.VMEM((NGATHER, IDX_W), jnp.int32),
        pltpu.VMEM((CHUNK * L, EMB), jnp.float32),
        pltpu.VMEM((CHUNK, EMB), jnp.float32),
        pltpu.VMEM((CHUNK,), jnp.int32),
        pltpu.VMEM((CHUNK, EMB), jnp.float32),
        pltpu.SemaphoreType.DMA,
        pltpu.SemaphoreType.DMA,
    ],
    compiler_params=pltpu.CompilerParams(use_tc_tiling_on_sc=False),
)
def _pool_sc(nids2d, click, emb, nid_emb, pooled_hbm, clicked_hbm,
             idx2, rows_v, pooled_v, click_v, yrows_v, gsem, csem):
    wid = lax.axis_index("s") * NC + lax.axis_index("c")
    base = wid * ROWS_PER_W
    offs = [0, 16, 32, 48, 64, 80, IDX_W - 16]

    def chunk_body(c, carry):
        row0 = pl.multiple_of(base + c * CHUNK, CHUNK)
        pltpu.sync_copy(
            nids2d.at[pl.ds(pl.multiple_of(row0 * L // IDX_W, 8), NGATHER)],
            idx2)
        pltpu.sync_copy(click.at[pl.ds(row0, CHUNK)], click_v)
        click_v[pl.ds(0, 16)] = _scramble(click_v[pl.ds(0, 16)])
        cp_click = pltpu.async_copy(nid_emb.at[click_v], yrows_v, csem)
        for g in range(NGATHER):
            vals = [_scramble(idx2[g, pl.ds(o, 16)]) for o in offs]
            for o, v in zip(offs, vals):
                idx2[g, pl.ds(o, 16)] = v
        cps = [
            pltpu.async_copy(
                emb.at[idx2.at[g]],
                rows_v.at[pl.ds(g * IDX_W, IDX_W)],
                gsem,
            )
            for g in range(NGATHER)
        ]
        for cp in cps:
            cp.wait()

        def row_body(i, carry2):
            k0 = i * L
            acc0 = rows_v[k0, pl.ds(0, 16)]
            acc1 = rows_v[k0, pl.ds(16, 16)]
            for j in range(1, L):
                acc0 = acc0 + rows_v[k0 + j, pl.ds(0, 16)]
                acc1 = acc1 + rows_v[k0 + j, pl.ds(16, 16)]
            pooled_v[i, pl.ds(0, 16)] = acc0
            pooled_v[i, pl.ds(16, 16)] = acc1
            return carry2

        lax.fori_loop(0, CHUNK, row_body, 0)
        cp_click.wait()
        pltpu.sync_copy(pooled_v, pooled_hbm.at[pl.ds(row0, CHUNK)])
        pltpu.sync_copy(yrows_v, clicked_hbm.at[pl.ds(row0, CHUNK)])
        return carry

    lax.fori_loop(0, NCHUNK, chunk_body, 0)


BM = 2048


def _mlp_body(x1_ref, x2_ref, w1_ref, b1_ref, w2_ref, b2_ref,
              w3_ref, b3_ref, w4_ref, b4_ref, y_ref):
    x = jnp.concatenate([x1_ref[...], x2_ref[...]], axis=1)
    dn = (((1,), (1,)), ((), ()))
    h = lax.dot_general(x, w1_ref[...], dn, preferred_element_type=jnp.float32)
    h = jnp.maximum(h + b1_ref[...], 0.0)
    h = lax.dot_general(h, w2_ref[...], dn, preferred_element_type=jnp.float32)
    h = jnp.maximum(h + b2_ref[...], 0.0)
    h = lax.dot_general(h, w3_ref[...], dn, preferred_element_type=jnp.float32)
    h = jnp.maximum(h + b3_ref[...], 0.0)
    h = lax.dot_general(h, w4_ref[...], dn, preferred_element_type=jnp.float32)
    y_ref[...] = jnp.maximum(h + b4_ref[...], 0.0)


def _mlp(x1, x2, W1, b1, W2, b2, W3, b3, W4, b4):
    full = lambda s: pl.BlockSpec(s, lambda i: (0, 0))
    return pl.pallas_call(
        _mlp_body,
        grid=(B // BM,),
        in_specs=[
            pl.BlockSpec((BM, EMB), lambda i: (i, 0)),
            pl.BlockSpec((BM, EMB), lambda i: (i, 0)),
            full(W1.shape), full((1, 256)),
            full(W2.shape), full((1, 256)),
            full(W3.shape), full((1, 128)),
            full(W4.shape), full((1, 2)),
        ],
        out_specs=pl.BlockSpec((BM, 2), lambda i: (i, 0)),
        out_shape=jax.ShapeDtypeStruct((B, 2), jnp.float32),
    )(x1, x2, W1, b1.reshape(1, -1), W2, b2.reshape(1, -1),
      W3, b3.reshape(1, -1), W4, b4.reshape(1, -1))


def kernel(input_nids, click_items, input_emb, nid_emb,
           W1, b1, W2, b2, W3, b3, W4, b4):
    emb_rm = _to_row_major(input_emb.T)
    nid_rm = _to_row_major(nid_emb.T)
    nids2d = input_nids.reshape(B * L // IDX_W, IDX_W)
    pooled, clicked = _pool_sc(nids2d, click_items, emb_rm, nid_rm)
    return _mlp(pooled, clicked, W1, b1, W2, b2, W3, b3, W4, b4)


# speedup vs baseline: 2.1771x; 1.2749x over previous
import functools

import jax
import jax.numpy as jnp
from jax import lax
from jax.experimental import pallas as pl
from jax.experimental.pallas import tpu as pltpu
from jax.experimental.pallas import tpu_sc as plsc

B = 16384
L = 50
EMB = 32
VOCAB = 1000000

NC = 2
NS = 16
NW = NC * NS
ROWS_PER_W = B // NW
CHUNK = 16
NCHUNK = ROWS_PER_W // CHUNK
IDX_W = 100
NGATHER = CHUNK * L // IDX_W

_sc_mesh = plsc.VectorSubcoreMesh(core_axis_name="c", subcore_axis_name="s")


TCOL = 8192
NBAND = TCOL // 512
NBLK = (VOCAB + TCOL - 1) // TCOL
VROWS = NBLK * TCOL


def _transpose_body(x_ref, o_ref):
    xt = x_ref[...].T
    bands = [xt[q * 512:(q + 1) * 512, :] for q in range(NBAND)]
    o_ref[...] = jnp.concatenate(bands, axis=1)


def _to_row_major(table_t):
    out = pl.pallas_call(
        _transpose_body,
        grid=(NBLK,),
        in_specs=[pl.BlockSpec((EMB, TCOL), lambda i: (0, i))],
        out_specs=pl.BlockSpec((512, NBAND * EMB), lambda i: (i, 0)),
        out_shape=jax.ShapeDtypeStruct((NBLK * 512, NBAND * EMB), jnp.float32),
    )(table_t)
    return out.reshape(VROWS, EMB)


def _scramble(idx):
    return ((idx >> 13) << 13) + ((idx & 511) << 4) + ((idx & 8191) >> 9)


@functools.partial(
    pl.kernel,
    mesh=_sc_mesh,
    out_type=[
        jax.ShapeDtypeStruct((B, EMB), jnp.float32),
        jax.ShapeDtypeStruct((B, EMB), jnp.float32),
    ],
    scratch_types=[
        pltpu.VMEM((NGATHER, IDX_W), jnp.int32),
        pltpu.VMEM((CHUNK * L, EMB), jnp.float32),
        pltpu.VMEM((CHUNK, EMB), jnp.float32),
        pltpu.VMEM((CHUNK,), jnp.int32),
        pltpu.VMEM((CHUNK, EMB), jnp.float32),
        pltpu.SemaphoreType.DMA,
        pltpu.SemaphoreType.DMA,
    ],
    compiler_params=pltpu.CompilerParams(use_tc_tiling_on_sc=False),
)
def _pool_sc(nids2d, click, emb, nid_emb, pooled_hbm, clicked_hbm,
             idx2, rows_v, pooled_v, click_v, yrows_v, gsem, csem):
    wid = lax.axis_index("s") * NC + lax.axis_index("c")
    base = wid * ROWS_PER_W
    offs = [0, 16, 32, 48, 64, 80, IDX_W - 16]

    def chunk_body(c, carry):
        row0 = pl.multiple_of(base + c * CHUNK, CHUNK)
        pltpu.sync_copy(
            nids2d.at[pl.ds(pl.multiple_of(row0 * L // IDX_W, 8), NGATHER)],
            idx2)
        pltpu.sync_copy(click.at[pl.ds(row0, CHUNK)], click_v)
        click_v[pl.ds(0, 16)] = _scramble(click_v[pl.ds(0, 16)])
        cp_click = pltpu.async_copy(nid_emb.at[click_v], yrows_v, csem)
        for g in range(NGATHER):
            vals = [_scramble(idx2[g, pl.ds(o, 16)]) for o in offs]
            for o, v in zip(offs, vals):
                idx2[g, pl.ds(o, 16)] = v
        cps = [
            pltpu.async_copy(
                emb.at[idx2.at[g]],
                rows_v.at[pl.ds(g * IDX_W, IDX_W)],
                gsem,
            )
            for g in range(NGATHER)
        ]
        for cp in cps:
            cp.wait()

        def row_body(i, carry2):
            k0 = i * L
            acc0 = rows_v[k0, pl.ds(0, 16)]
            acc1 = rows_v[k0, pl.ds(16, 16)]
            for j in range(1, L):
                acc0 = acc0 + rows_v[k0 + j, pl.ds(0, 16)]
                acc1 = acc1 + rows_v[k0 + j, pl.ds(16, 16)]
            pooled_v[i, pl.ds(0, 16)] = acc0
            pooled_v[i, pl.ds(16, 16)] = acc1
            return carry2

        lax.fori_loop(0, CHUNK, row_body, 0)
        cp_click.wait()
        pltpu.sync_copy(pooled_v, pooled_hbm.at[pl.ds(row0, CHUNK)])
        pltpu.sync_copy(yrows_v, clicked_hbm.at[pl.ds(row0, CHUNK)])
        return carry

    lax.fori_loop(0, NCHUNK, chunk_body, 0)


BM = 2048


def _mlp_body(x1_ref, x2_ref, w1_ref, b1_ref, w2_ref, b2_ref,
              w3_ref, b3_ref, w4_ref, b4_ref, y_ref):
    x = jnp.concatenate([x1_ref[...], x2_ref[...]], axis=1)
    dn = (((1,), (1,)), ((), ()))
    h = lax.dot_general(x, w1_ref[...], dn, preferred_element_type=jnp.float32)
    h = jnp.maximum(h + b1_ref[...], 0.0)
    h = lax.dot_general(h, w2_ref[...], dn, preferred_element_type=jnp.float32)
    h = jnp.maximum(h + b2_ref[...], 0.0)
    h = lax.dot_general(h, w3_ref[...], dn, preferred_element_type=jnp.float32)
    h = jnp.maximum(h + b3_ref[...], 0.0)
    h = lax.dot_general(h, w4_ref[...], dn, preferred_element_type=jnp.float32)
    y_ref[...] = jnp.maximum(h + b4_ref[...], 0.0)


def _mlp(x1, x2, W1, b1, W2, b2, W3, b3, W4, b4):
    full = lambda s: pl.BlockSpec(s, lambda i: (0, 0))
    return pl.pallas_call(
        _mlp_body,
        grid=(B // BM,),
        in_specs=[
            pl.BlockSpec((BM, EMB), lambda i: (i, 0)),
            pl.BlockSpec((BM, EMB), lambda i: (i, 0)),
            full(W1.shape), full((1, 256)),
            full(W2.shape), full((1, 256)),
            full(W3.shape), full((1, 128)),
            full(W4.shape), full((1, 2)),
        ],
        out_specs=pl.BlockSpec((BM, 2), lambda i: (i, 0)),
        out_shape=jax.ShapeDtypeStruct((B, 2), jnp.float32),
    )(x1, x2, W1, b1.reshape(1, -1), W2, b2.reshape(1, -1),
      W3, b3.reshape(1, -1), W4, b4.reshape(1, -1))


def kernel(input_nids, click_items, input_emb, nid_emb,
           W1, b1, W2, b2, W3, b3, W4, b4):
    emb_rm = _to_row_major(input_emb.T)
    nid_rm = _to_row_major(nid_emb.T)
    nids2d = input_nids.reshape(B * L // IDX_W, IDX_W)
    pooled, clicked = _pool_sc(nids2d, click_items, emb_rm, nid_rm)
    return _mlp(pooled, clicked, W1, b1, W2, b2, W3, b3, W4, b4)


# speedup vs baseline: 2.2080x; 1.0142x over previous
import functools

import jax
import jax.numpy as jnp
from jax import lax
from jax.experimental import pallas as pl
from jax.experimental.pallas import tpu as pltpu
from jax.experimental.pallas import tpu_sc as plsc

B = 16384
L = 50
EMB = 32
VOCAB = 1000000

NC = 2
NS = 16
NW = NC * NS
ROWS_PER_W = B // NW
CHUNK = 16
NCHUNK = ROWS_PER_W // CHUNK
IDX_W = 100
NGATHER = CHUNK * L // IDX_W

_sc_mesh = plsc.VectorSubcoreMesh(core_axis_name="c", subcore_axis_name="s")


TCOL = 8192
NBAND = TCOL // 512
NBLK = (VOCAB + TCOL - 1) // TCOL
VROWS = NBLK * TCOL


def _transpose_body(x_ref, o_ref):
    xt = x_ref[...].T
    bands = [xt[q * 512:(q + 1) * 512, :] for q in range(NBAND)]
    o_ref[...] = jnp.concatenate(bands, axis=1)


def _to_row_major(table_t):
    out = pl.pallas_call(
        _transpose_body,
        grid=(NBLK,),
        in_specs=[pl.BlockSpec((EMB, TCOL), lambda i: (0, i))],
        out_specs=pl.BlockSpec((512, NBAND * EMB), lambda i: (i, 0)),
        out_shape=jax.ShapeDtypeStruct((NBLK * 512, NBAND * EMB), jnp.float32),
    )(table_t)
    return out.reshape(VROWS, EMB)


def _scramble(idx):
    return ((idx >> 13) << 13) + ((idx & 511) << 4) + ((idx & 8191) >> 9)


@functools.partial(
    pl.kernel,
    mesh=_sc_mesh,
    out_type=jax.ShapeDtypeStruct((B, EMB), jnp.float32),
    scratch_types=[
        pltpu.VMEM((NGATHER, IDX_W), jnp.int32),
        pltpu.VMEM((CHUNK * L, EMB), jnp.float32),
        pltpu.VMEM((CHUNK, EMB), jnp.float32),
        pltpu.SemaphoreType.DMA,
    ],
    compiler_params=pltpu.CompilerParams(use_tc_tiling_on_sc=False),
)
def _pool_sc(nids2d, emb, pooled_hbm, idx2, rows_v, pooled_v, gsem):
    wid = lax.axis_index("s") * NC + lax.axis_index("c")
    base = wid * ROWS_PER_W
    offs = [0, 16, 32, 48, 64, 80, IDX_W - 16]

    def chunk_body(c, carry):
        row0 = pl.multiple_of(base + c * CHUNK, CHUNK)
        pltpu.sync_copy(
            nids2d.at[pl.ds(pl.multiple_of(row0 * L // IDX_W, 8), NGATHER)],
            idx2)
        for g in range(NGATHER):
            vals = [_scramble(idx2[g, pl.ds(o, 16)]) for o in offs]
            for o, v in zip(offs, vals):
                idx2[g, pl.ds(o, 16)] = v
        cps = [
            pltpu.async_copy(
                emb.at[idx2.at[g]],
                rows_v.at[pl.ds(g * IDX_W, IDX_W)],
                gsem,
            )
            for g in range(NGATHER)
        ]
        for cp in cps:
            cp.wait()

        def row_body(i, carry2):
            k0 = i * L
            acc0 = rows_v[k0, pl.ds(0, 16)]
            acc1 = rows_v[k0, pl.ds(16, 16)]
            for j in range(1, L):
                acc0 = acc0 + rows_v[k0 + j, pl.ds(0, 16)]
                acc1 = acc1 + rows_v[k0 + j, pl.ds(16, 16)]
            pooled_v[i, pl.ds(0, 16)] = acc0
            pooled_v[i, pl.ds(16, 16)] = acc1
            return carry2

        lax.fori_loop(0, CHUNK, row_body, 0)
        pltpu.sync_copy(pooled_v, pooled_hbm.at[pl.ds(row0, CHUNK)])
        return carry

    lax.fori_loop(0, NCHUNK, chunk_body, 0)


CLK = 128


@functools.partial(
    pl.kernel,
    mesh=_sc_mesh,
    out_type=jax.ShapeDtypeStruct((B, EMB), jnp.float32),
    scratch_types=[
        pltpu.VMEM((CLK,), jnp.int32),
        pltpu.VMEM((CLK, EMB), jnp.float32),
        pltpu.SemaphoreType.DMA,
    ],
    compiler_params=pltpu.CompilerParams(use_tc_tiling_on_sc=False),
)
def _click_sc(click, nid_emb, out_hbm, cidx, crows, csem):
    wid = lax.axis_index("s") * NC + lax.axis_index("c")
    base = wid * ROWS_PER_W

    def chunk_body(c, carry):
        row0 = pl.multiple_of(base + c * CLK, CLK)
        pltpu.sync_copy(click.at[pl.ds(row0, CLK)], cidx)
        for k in range(CLK // 16):
            cidx[pl.ds(k * 16, 16)] = _scramble(cidx[pl.ds(k * 16, 16)])
        pltpu.async_copy(nid_emb.at[cidx], crows, csem).wait()
        pltpu.sync_copy(crows, out_hbm.at[pl.ds(row0, CLK)])
        return carry

    lax.fori_loop(0, ROWS_PER_W // CLK, chunk_body, 0)


BM = 2048


def _mlp_body(x1_ref, x2_ref, w1_ref, b1_ref, w2_ref, b2_ref,
              w3_ref, b3_ref, w4_ref, b4_ref, y_ref):
    x = jnp.concatenate([x1_ref[...], x2_ref[...]], axis=1)
    dn = (((1,), (1,)), ((), ()))
    h = lax.dot_general(x, w1_ref[...], dn, preferred_element_type=jnp.float32)
    h = jnp.maximum(h + b1_ref[...], 0.0)
    h = lax.dot_general(h, w2_ref[...], dn, preferred_element_type=jnp.float32)
    h = jnp.maximum(h + b2_ref[...], 0.0)
    h = lax.dot_general(h, w3_ref[...], dn, preferred_element_type=jnp.float32)
    h = jnp.maximum(h + b3_ref[...], 0.0)
    h = lax.dot_general(h, w4_ref[...], dn, preferred_element_type=jnp.float32)
    y_ref[...] = jnp.maximum(h + b4_ref[...], 0.0)


def _mlp(x1, x2, W1, b1, W2, b2, W3, b3, W4, b4):
    full = lambda s: pl.BlockSpec(s, lambda i: (0, 0))
    return pl.pallas_call(
        _mlp_body,
        grid=(B // BM,),
        in_specs=[
            pl.BlockSpec((BM, EMB), lambda i: (i, 0)),
            pl.BlockSpec((BM, EMB), lambda i: (i, 0)),
            full(W1.shape), full((1, 256)),
            full(W2.shape), full((1, 256)),
            full(W3.shape), full((1, 128)),
            full(W4.shape), full((1, 2)),
        ],
        out_specs=pl.BlockSpec((BM, 2), lambda i: (i, 0)),
        out_shape=jax.ShapeDtypeStruct((B, 2), jnp.float32),
    )(x1, x2, W1, b1.reshape(1, -1), W2, b2.reshape(1, -1),
      W3, b3.reshape(1, -1), W4, b4.reshape(1, -1))


def kernel(input_nids, click_items, input_emb, nid_emb,
           W1, b1, W2, b2, W3, b3, W4, b4):
    emb_rm = _to_row_major(input_emb.T)
    nid_rm = _to_row_major(nid_emb.T)
    nids2d = input_nids.reshape(B * L // IDX_W, IDX_W)
    pooled = _pool_sc(nids2d, emb_rm)
    clicked = _click_sc(click_items, nid_rm)
    return _mlp(pooled, clicked, W1, b1, W2, b2, W3, b3, W4, b4)
